# Initial kernel scaffold; baseline (speedup 1.0000x reference)
import functools

import jax
import jax.numpy as jnp
from jax import lax
from jax.experimental import pallas as pl
from jax.experimental.pallas import tpu as pltpu
from jax.experimental.pallas import tpu_sc as plsc

_NC = 2
_NS = 16
_NW = _NC * _NS
_CHUNK = 128


def kernel(input_, weight):
    B, H = input_.shape
    V, D = weight.shape
    N = B * H
    assert N % (_NW * _CHUNK) == 0
    n_chunks = N // (_NW * _CHUNK)
    idx = input_.reshape(N // _CHUNK, _CHUNK).astype(jnp.int32)

    mesh = plsc.VectorSubcoreMesh(core_axis_name="c", subcore_axis_name="s")

    @functools.partial(
        pl.kernel,
        out_type=jax.ShapeDtypeStruct((N, D), jnp.float32),
        mesh=mesh,
        scratch_types=[
            pltpu.VMEM((n_chunks, _CHUNK), jnp.int32),
            pltpu.VMEM((_CHUNK, D), jnp.float32),
            pltpu.SemaphoreType.DMA,
        ],
    )
    def emb(idx_hbm, w_hbm, out_hbm, idx_v, buf, gsem):
        wid = lax.axis_index("s") * _NC + lax.axis_index("c")
        row0 = wid * n_chunks
        pltpu.sync_copy(idx_hbm.at[pl.ds(row0, n_chunks)], idx_v)

        @pl.loop(0, n_chunks)
        def _body(j):
            pltpu.async_copy(w_hbm.at[idx_v.at[j]], buf, gsem).wait()
            pltpu.sync_copy(buf, out_hbm.at[pl.ds((row0 + j) * _CHUNK, _CHUNK)])

    out = emb(idx, weight)
    return out.reshape(B, H, D)


# speedup vs baseline: 1.6821x; 1.6821x over previous
import functools

import jax
import jax.numpy as jnp
from jax import lax
from jax.experimental import pallas as pl
from jax.experimental.pallas import tpu as pltpu
from jax.experimental.pallas import tpu_sc as plsc

_NC = 2
_NS = 16
_NW = _NC * _NS
_CHUNK = 128


def kernel(input_, weight):
    B, H = input_.shape
    V, D = weight.shape
    N = B * H
    assert N % (_NW * _CHUNK) == 0
    n_chunks = N // (_NW * _CHUNK)
    idx = input_.reshape(N // _CHUNK, _CHUNK).astype(jnp.int32)

    mesh = plsc.VectorSubcoreMesh(core_axis_name="c", subcore_axis_name="s")

    @functools.partial(
        pl.kernel,
        out_type=jax.ShapeDtypeStruct((N, D), jnp.float32),
        mesh=mesh,
        scratch_types=[
            pltpu.VMEM((n_chunks, _CHUNK), jnp.int32),
            pltpu.VMEM((_CHUNK, D), jnp.float32),
            pltpu.SemaphoreType.DMA,
        ],
        compiler_params=pltpu.CompilerParams(use_tc_tiling_on_sc=False),
    )
    def emb(idx_hbm, w_hbm, out_hbm, idx_v, buf, gsem):
        wid = lax.axis_index("s") * _NC + lax.axis_index("c")
        row0 = wid * n_chunks
        pltpu.sync_copy(idx_hbm.at[pl.ds(row0, n_chunks)], idx_v)

        @pl.loop(0, n_chunks)
        def _body(j):
            pltpu.async_copy(w_hbm.at[idx_v.at[j]], buf, gsem).wait()
            pltpu.sync_copy(buf, out_hbm.at[pl.ds((row0 + j) * _CHUNK, _CHUNK)])

    out = emb(idx, weight)
    return out.reshape(B, H, D)


# speedup vs baseline: 1.8687x; 1.1110x over previous
import functools

import jax
import jax.numpy as jnp
from jax import lax
from jax.experimental import pallas as pl
from jax.experimental.pallas import tpu as pltpu
from jax.experimental.pallas import tpu_sc as plsc

_NC = 2
_NS = 16
_NW = _NC * _NS
_CHUNK = 128
_NBUF = 4


def kernel(input_, weight):
    B, H = input_.shape
    V, D = weight.shape
    N = B * H
    assert N % (_NW * _CHUNK) == 0
    n_chunks = N // (_NW * _CHUNK)
    assert n_chunks % _NBUF == 0 and n_chunks >= 2 * _NBUF
    idx = input_.reshape(N // _CHUNK, _CHUNK).astype(jnp.int32)

    mesh = plsc.VectorSubcoreMesh(core_axis_name="c", subcore_axis_name="s")

    @functools.partial(
        pl.kernel,
        out_type=jax.ShapeDtypeStruct((N, D), jnp.float32),
        mesh=mesh,
        scratch_types=[
            pltpu.VMEM((n_chunks, _CHUNK), jnp.int32),
            [pltpu.VMEM((_CHUNK, D), jnp.float32) for _ in range(_NBUF)],
            [pltpu.SemaphoreType.DMA for _ in range(_NBUF)],
            [pltpu.SemaphoreType.DMA for _ in range(_NBUF)],
        ],
        compiler_params=pltpu.CompilerParams(use_tc_tiling_on_sc=False),
    )
    def emb(idx_hbm, w_hbm, out_hbm, idx_v, bufs, gsems, wsems):
        wid = lax.axis_index("s") * _NC + lax.axis_index("c")
        row0 = wid * n_chunks
        pltpu.sync_copy(idx_hbm.at[pl.ds(row0, n_chunks)], idx_v)

        def gather_desc(j, b):
            return pltpu.make_async_copy(w_hbm.at[idx_v.at[j]], bufs[b], gsems[b])

        def write_desc(j, b):
            return pltpu.make_async_copy(
                bufs[b], out_hbm.at[pl.ds((row0 + j) * _CHUNK, _CHUNK)], wsems[b])

        for b in range(_NBUF):
            gather_desc(b, b).start()

        @pl.loop(0, n_chunks - _NBUF, step=_NBUF)
        def _body(j):
            for b in range(_NBUF):
                gather_desc(j + b, b).wait()
                write_desc(j + b, b).start()
            for b in range(_NBUF):
                write_desc(j + b, b).wait()
                gather_desc(j + _NBUF + b, b).start()

        j_last = n_chunks - _NBUF
        for b in range(_NBUF):
            gather_desc(j_last + b, b).wait()
            write_desc(j_last + b, b).start()
        for b in range(_NBUF):
            write_desc(j_last + b, b).wait()

    out = emb(idx, weight)
    return out.reshape(B, H, D)


# speedup vs baseline: 1.8739x; 1.0028x over previous
import functools

import jax
import jax.numpy as jnp
from jax import lax
from jax.experimental import pallas as pl
from jax.experimental.pallas import tpu as pltpu
from jax.experimental.pallas import tpu_sc as plsc

_NC = 2
_NS = 16
_NW = _NC * _NS
_CHUNK = 128
_NBUF = 8


def kernel(input_, weight):
    B, H = input_.shape
    V, D = weight.shape
    N = B * H
    assert N % (_NW * _CHUNK) == 0
    n_chunks = N // (_NW * _CHUNK)
    assert n_chunks % _NBUF == 0 and n_chunks >= 2 * _NBUF
    idx = input_.reshape(N // _CHUNK, _CHUNK).astype(jnp.int32)

    mesh = plsc.VectorSubcoreMesh(core_axis_name="c", subcore_axis_name="s")

    @functools.partial(
        pl.kernel,
        out_type=jax.ShapeDtypeStruct((N, D), jnp.float32),
        mesh=mesh,
        scratch_types=[
            pltpu.VMEM((n_chunks, _CHUNK), jnp.int32),
            [pltpu.VMEM((_CHUNK, D), jnp.float32) for _ in range(_NBUF)],
            [pltpu.SemaphoreType.DMA for _ in range(_NBUF)],
            [pltpu.SemaphoreType.DMA for _ in range(_NBUF)],
        ],
        compiler_params=pltpu.CompilerParams(use_tc_tiling_on_sc=False),
    )
    def emb(idx_hbm, w_hbm, out_hbm, idx_v, bufs, gsems, wsems):
        wid = lax.axis_index("s") * _NC + lax.axis_index("c")
        row0 = wid * n_chunks
        pltpu.sync_copy(idx_hbm.at[pl.ds(row0, n_chunks)], idx_v)

        def gather_desc(j, b):
            return pltpu.make_async_copy(w_hbm.at[idx_v.at[j]], bufs[b], gsems[b])

        def write_desc(j, b):
            return pltpu.make_async_copy(
                bufs[b], out_hbm.at[pl.ds((row0 + j) * _CHUNK, _CHUNK)], wsems[b])

        for b in range(_NBUF):
            gather_desc(b, b).start()

        @pl.loop(0, n_chunks - _NBUF, step=_NBUF)
        def _body(j):
            for b in range(_NBUF):
                gather_desc(j + b, b).wait()
                write_desc(j + b, b).start()
            for b in range(_NBUF):
                write_desc(j + b, b).wait()
                gather_desc(j + _NBUF + b, b).start()

        j_last = n_chunks - _NBUF
        for b in range(_NBUF):
            gather_desc(j_last + b, b).wait()
            write_desc(j_last + b, b).start()
        for b in range(_NBUF):
            write_desc(j_last + b, b).wait()

    out = emb(idx, weight)
    return out.reshape(B, H, D)


# speedup vs baseline: 2.8287x; 1.5095x over previous
import functools

import jax
import jax.numpy as jnp
from jax import lax
from jax.experimental import pallas as pl
from jax.experimental.pallas import tpu as pltpu
from jax.experimental.pallas import tpu_sc as plsc

_NC = 2
_NS = 16


def kernel(input_, weight):
    B, H = input_.shape
    V, D = weight.shape
    BH = B // 2
    NU = 2 * H
    n_u = (NU + _NS - 1) // _NS
    d_per_c = D // _NC
    idx2 = input_.T.reshape(NU, BH)
    wT = weight.T

    mesh = plsc.VectorSubcoreMesh(core_axis_name="c", subcore_axis_name="s")

    @functools.partial(
        pl.kernel,
        out_type=jax.ShapeDtypeStruct((H * D, B), jnp.float32),
        mesh=mesh,
        scratch_types=[
            pltpu.VMEM_SHARED((V,), jnp.float32),
            [pltpu.VMEM((BH,), jnp.int32) for _ in range(n_u)],
            pltpu.VMEM((BH,), jnp.float32),
            pltpu.SemaphoreType.DMA,
        ],
    )
    def emb(idx_hbm, wT_hbm, out_hbm, col, idx_v, dst_v, gsem):
        c = lax.axis_index("c")
        s = lax.axis_index("s")
        d0 = c * d_per_c

        for k in range(n_u):
            u = s + k * _NS
            @pl.when(u < NU)
            def _load(k=k, u=u):
                pltpu.sync_copy(idx_hbm.at[u], idx_v[k])

        @pl.loop(0, d_per_c)
        def _body(j):
            @pl.when(s == 0)
            def _stage():
                pltpu.sync_copy(wT_hbm.at[d0 + j], col)
            plsc.subcore_barrier()
            for k in range(n_u):
                u = s + k * _NS
                @pl.when(u < NU)
                def _one(k=k, u=u):
                    h = u // 2
                    bh = u % 2
                    pltpu.async_copy(col.at[idx_v[k]], dst_v, gsem).wait()
                    boff = pl.multiple_of(bh * BH, BH)
                    pltpu.sync_copy(dst_v, out_hbm.at[h * D + d0 + j, pl.ds(boff, BH)])
            plsc.subcore_barrier()

    out2 = emb(idx2, wT)
    return out2.reshape(H, D, B).transpose(2, 0, 1)


# speedup vs baseline: 3.2051x; 1.1331x over previous
import functools

import jax
import jax.numpy as jnp
from jax import lax
from jax.experimental import pallas as pl
from jax.experimental.pallas import tpu as pltpu
from jax.experimental.pallas import tpu_sc as plsc

_NC = 2
_NS = 16


def kernel(input_, weight):
    B, H = input_.shape
    V, D = weight.shape
    BH = B // 4
    NU = 4 * H
    n_u = (NU + _NS - 1) // _NS
    d_per_c = D // _NC
    idx2 = input_.T.reshape(NU, BH)
    wT = weight.T

    mesh = plsc.VectorSubcoreMesh(core_axis_name="c", subcore_axis_name="s")

    @functools.partial(
        pl.kernel,
        out_type=jax.ShapeDtypeStruct((H * D, B), jnp.float32),
        mesh=mesh,
        scratch_types=[
            pltpu.VMEM_SHARED((V,), jnp.float32),
            [pltpu.VMEM((BH,), jnp.int32) for _ in range(n_u)],
            [pltpu.VMEM((BH,), jnp.float32) for _ in range(2)],
            [pltpu.SemaphoreType.DMA for _ in range(2)],
        ],
    )
    def emb(idx_hbm, wT_hbm, out_hbm, col, idx_v, dst_v, gsem):
        def gat(k):
            return pltpu.make_async_copy(col.at[idx_v[k]], dst_v[k % 2], gsem[k % 2])

        c = lax.axis_index("c")
        s = lax.axis_index("s")
        d0 = c * d_per_c

        for k in range(n_u):
            u = s + k * _NS
            @pl.when(u < NU)
            def _load(k=k, u=u):
                pltpu.sync_copy(idx_hbm.at[u], idx_v[k])

        @pl.loop(0, d_per_c)
        def _body(j):
            @pl.when(s == 0)
            def _stage():
                pltpu.sync_copy(wT_hbm.at[d0 + j], col)
            plsc.subcore_barrier()
            gat(0).start()
            for k in range(n_u):
                u = s + k * _NS
                if k + 1 < n_u:
                    un = s + (k + 1) * _NS
                    @pl.when(un < NU)
                    def _nxt(k=k):
                        gat(k + 1).start()
                @pl.when(u < NU)
                def _one(k=k, u=u):
                    h = u // 4
                    bh = u % 4
                    gat(k).wait()
                    boff = pl.multiple_of(bh * BH, BH)
                    pltpu.sync_copy(dst_v[k % 2], out_hbm.at[h * D + d0 + j, pl.ds(boff, BH)])
            plsc.subcore_barrier()

    out2 = emb(idx2, wT)
    return out2.reshape(H, D, B).transpose(2, 0, 1)
